# baseline (speedup 1.0000x reference)
import functools
import math

import jax
import jax.numpy as jnp
from jax import lax
from jax.experimental import pallas as pl
from jax.experimental.pallas import tpu as pltpu
from jax.experimental.pallas import tpu_sc as plsc

EMBEDDING_DIM = 1024
PADDING_IDX = 0

ROW_BLOCK = 256
SEED_ROWS = 64
SC_CHUNK = 64


def _table_kernel(inv_freq_ref, out_ref, base_ref):
    i = pl.program_id(0)
    half = inv_freq_ref.shape[1]
    rows = out_ref.shape[0]
    w = inv_freq_ref[0, :][None, :]

    @pl.when(i == 0)
    def _init_base():
        dp = jax.lax.broadcasted_iota(
            jnp.int32, (SEED_ROWS, 1), 0
        ).astype(jnp.float32)
        d = dp * w
        base_ref[:SEED_ROWS, :half] = jnp.sin(d)
        base_ref[:SEED_ROWS, half:] = jnp.cos(d)
        n = SEED_ROWS
        while n < rows:
            rb = float(n) * w
            srb = jnp.sin(rb)
            crb = jnp.cos(rb)
            s_lo = base_ref[:n, :half]
            c_lo = base_ref[:n, half:]
            base_ref[n : 2 * n, :half] = s_lo * crb + c_lo * srb
            base_ref[n : 2 * n, half:] = c_lo * crb - s_lo * srb
            n *= 2

    sd = base_ref[:, :half]
    cd = base_ref[:, half:]
    b = (i * rows).astype(jnp.float32) * w
    sb = jnp.sin(b)
    cb = jnp.cos(b)
    tile_sin = sd * cb + cd * sb
    tile_cos = cd * cb - sd * sb
    out_ref[...] = jnp.concatenate([tile_sin, tile_cos], axis=1)

    @pl.when(i == 0)
    def _zero_pad_row():
        out_ref[PADDING_IDX : PADDING_IDX + 1, :] = jnp.zeros(
            (1, out_ref.shape[1]), jnp.float32
        )


def _make_table(seq_len):
    half_dim = EMBEDDING_DIM // 2
    scale = math.log(10000.0) / (half_dim - 1)
    inv_freq = jnp.exp(
        jnp.arange(half_dim, dtype=jnp.float32) * -scale
    ).reshape(1, half_dim)
    n_blocks = seq_len // ROW_BLOCK
    return pl.pallas_call(
        _table_kernel,
        grid=(n_blocks,),
        in_specs=[pl.BlockSpec((1, half_dim), lambda i: (0, 0))],
        out_specs=pl.BlockSpec((ROW_BLOCK, EMBEDDING_DIM), lambda i: (i, 0)),
        out_shape=jax.ShapeDtypeStruct((seq_len, EMBEDDING_DIM), jnp.float32),
        scratch_shapes=[pltpu.VMEM((ROW_BLOCK, EMBEDDING_DIM), jnp.float32)],
    )(inv_freq)


def _sc_broadcast(table, bsz):
    seq_len, dim = table.shape
    info = plsc.get_sparse_core_info()
    nc, ns = int(info.num_cores), int(info.num_subcores)
    nw = nc * ns
    rows_per_w = seq_len // nw
    n_chunks = rows_per_w // SC_CHUNK
    mesh = plsc.VectorSubcoreMesh(core_axis_name="c", subcore_axis_name="s")

    @functools.partial(
        pl.kernel,
        mesh=mesh,
        out_type=jax.ShapeDtypeStruct((bsz * seq_len, dim), jnp.float32),
        scratch_types=[pltpu.VMEM((SC_CHUNK, dim), jnp.float32)],
    )
    def sc_copy(table_hbm, out_hbm, rows_v):
        wid = lax.axis_index("s") * nc + lax.axis_index("c")
        base = wid * rows_per_w
        for j in range(n_chunks):
            r0 = base + j * SC_CHUNK
            pltpu.sync_copy(table_hbm.at[pl.ds(r0, SC_CHUNK)], rows_v)
            for b in range(bsz):
                pltpu.sync_copy(
                    rows_v, out_hbm.at[pl.ds(b * seq_len + r0, SC_CHUNK)]
                )

    return sc_copy(table)


def kernel(input):
    bsz, seq_len = input.shape
    table = _make_table(seq_len)
    flat = _sc_broadcast(table, bsz)
    return flat.reshape(bsz, seq_len, EMBEDDING_DIM)


# speedup vs baseline: 2.2033x; 2.2033x over previous
import math

import jax
import jax.numpy as jnp
from jax.experimental import pallas as pl
from jax.experimental.pallas import tpu as pltpu

EMBEDDING_DIM = 1024
PADDING_IDX = 0

ROW_BLOCK = 256
SEED_ROWS = 64


def _sinusoid_kernel(inv_freq_ref, out_ref, base_ref):
    i = pl.program_id(0)
    half = inv_freq_ref.shape[1]
    rows = out_ref.shape[1]
    w = inv_freq_ref[0, :][None, :]

    @pl.when(i == 0)
    def _init_base():
        dp = jax.lax.broadcasted_iota(
            jnp.int32, (SEED_ROWS, 1), 0
        ).astype(jnp.float32)
        d = dp * w
        base_ref[:SEED_ROWS, :half] = jnp.sin(d)
        base_ref[:SEED_ROWS, half:] = jnp.cos(d)
        n = SEED_ROWS
        while n < rows:
            rb = float(n) * w
            srb = jnp.sin(rb)
            crb = jnp.cos(rb)
            s_lo = base_ref[:n, :half]
            c_lo = base_ref[:n, half:]
            base_ref[n : 2 * n, :half] = s_lo * crb + c_lo * srb
            base_ref[n : 2 * n, half:] = c_lo * crb - s_lo * srb
            n *= 2

    sd = base_ref[:, :half]
    cd = base_ref[:, half:]
    b = (i * rows).astype(jnp.float32) * w
    sb = jnp.sin(b)
    cb = jnp.cos(b)
    tile_sin = sd * cb + cd * sb
    tile_cos = cd * cb - sd * sb
    tile = jnp.concatenate([tile_sin, tile_cos], axis=1)
    out_ref[...] = jnp.broadcast_to(tile[None], out_ref.shape)

    @pl.when(i == 0)
    def _zero_pad_row():
        out_ref[:, PADDING_IDX : PADDING_IDX + 1, :] = jnp.zeros(
            (out_ref.shape[0], 1, out_ref.shape[2]), jnp.float32
        )


def kernel(input):
    bsz, seq_len = input.shape
    half_dim = EMBEDDING_DIM // 2
    scale = math.log(10000.0) / (half_dim - 1)
    inv_freq = jnp.exp(
        jnp.arange(half_dim, dtype=jnp.float32) * -scale
    ).reshape(1, half_dim)

    n_blocks = seq_len // ROW_BLOCK
    out = pl.pallas_call(
        _sinusoid_kernel,
        grid=(n_blocks,),
        in_specs=[
            pl.BlockSpec((1, half_dim), lambda i: (0, 0)),
        ],
        out_specs=pl.BlockSpec(
            (bsz, ROW_BLOCK, EMBEDDING_DIM), lambda i: (0, i, 0)
        ),
        out_shape=jax.ShapeDtypeStruct(
            (bsz, seq_len, EMBEDDING_DIM), jnp.float32
        ),
        scratch_shapes=[
            pltpu.VMEM((ROW_BLOCK, EMBEDDING_DIM), jnp.float32)
        ],
    )(inv_freq)
    return out
